# baseline (speedup 1.0000x reference)
import jax
import jax.numpy as jnp
from jax.experimental import pallas as pl
from jax.experimental.pallas import tpu as pltpu

_B, _S, _H = 16, 4096, 768
_MAX_ROWS, _MAX_COLS = 64, 32
_NUM_CELLS = _MAX_ROWS * _MAX_COLS
_NSPLIT = 4
_SUB = _S // _NSPLIT
_NEG = -10000.0
_EPS = 1e-10


def _body(*refs):
    x_refs = refs[:_NSPLIT]
    idxr_ref, idxc_ref, mask_ref, w_ref, b_ref, out_ref, acc_ref = refs[_NSPLIT:]

    w = w_ref[...]
    idx_row = idxr_ref[0]
    idx_col = idxc_ref[0]

    acc = jnp.zeros((_MAX_ROWS, 2 * _MAX_COLS), jnp.float32)
    for k in range(_NSPLIT):
        x = x_refs[k][0, 0]
        z = jnp.sum(x * w, axis=1, keepdims=True)
        hi = idx_row[:, k * _SUB:(k + 1) * _SUB] >> 5
        lo = idx_col[k * _SUB:(k + 1) * _SUB, :] & 31
        oh_hi = (jax.lax.broadcasted_iota(jnp.int32, (_MAX_ROWS, _SUB), 0)
                 == hi).astype(jnp.float32)
        oh_lo = (jax.lax.broadcasted_iota(jnp.int32, (_SUB, _MAX_COLS), 1)
                 == lo).astype(jnp.float32)
        rhs = jnp.concatenate([oh_lo * z, oh_lo], axis=1)
        acc = acc + jax.lax.dot(oh_hi, rhs,
                                preferred_element_type=jnp.float32)
    acc_ref[...] = acc

    sums = acc_ref[:, :_MAX_COLS]
    cnts = acc_ref[:, _MAX_COLS:]
    bias = b_ref[0, 0]
    cell_logits = jnp.where(cnts > 0.0,
                            sums / jnp.maximum(cnts, 1.0) + bias, 0.0)
    m = mask_ref[0]
    colsum = jnp.sum(cell_logits * m, axis=0, keepdims=True)
    colcnt = jnp.sum(m, axis=0, keepdims=True)
    col = colsum / (colcnt + _EPS)
    j = jax.lax.broadcasted_iota(jnp.int32, (1, _MAX_COLS), 1)
    pad = jnp.logical_and(colcnt < 0.5, j != 0)
    col = (col + _NEG * pad.astype(jnp.float32)
           + _NEG * (j == 0).astype(jnp.float32))
    out_ref[0] = col


def kernel(inputs, cell_index, cell_mask, column_output_weights,
           column_output_bias):
    x_split = inputs.reshape(_B, _NSPLIT, _SUB, _H)
    idx_row = cell_index.reshape(_B, 1, _S)
    idx_col = cell_index.reshape(_B, _S, 1)
    mask = cell_mask.reshape(_B, _MAX_ROWS, _MAX_COLS)
    w = column_output_weights.reshape(1, _H)
    b = jnp.reshape(column_output_bias, (1, 1)).astype(jnp.float32)

    def x_spec(k):
        return pl.BlockSpec((1, 1, _SUB, _H), lambda b_, k=k: (b_, k, 0, 0))

    return pl.pallas_call(
        _body,
        grid=(_B,),
        in_specs=[x_spec(k) for k in range(_NSPLIT)] + [
            pl.BlockSpec((1, 1, _S), lambda b_: (b_, 0, 0)),
            pl.BlockSpec((1, _S, 1), lambda b_: (b_, 0, 0)),
            pl.BlockSpec((1, _MAX_ROWS, _MAX_COLS), lambda b_: (b_, 0, 0)),
            pl.BlockSpec((1, _H), lambda b_: (0, 0)),
            pl.BlockSpec(memory_space=pltpu.SMEM),
        ],
        out_specs=pl.BlockSpec((1, 1, _MAX_COLS), lambda b_: (b_, 0, 0)),
        out_shape=jax.ShapeDtypeStruct((_B, 1, _MAX_COLS), jnp.float32),
        scratch_shapes=[
            pltpu.VMEM((_MAX_ROWS, 2 * _MAX_COLS), jnp.float32),
        ],
        compiler_params=pltpu.CompilerParams(
            dimension_semantics=("arbitrary",),
        ),
    )(*([x_split] * _NSPLIT + [idx_row, idx_col, mask, w, b])
      ).reshape(_B, _MAX_COLS)
